# baseline (speedup 1.0000x reference)
import functools

import jax
import jax.numpy as jnp
from jax import lax
from jax.experimental import pallas as pl
from jax.experimental.pallas import tpu as pltpu
from jax.experimental.pallas import tpu_sc as plsc

N = 10000
NP = 10112
EP = 172032
H1 = 8
O1 = 128
O2 = 64
D1A = 144
D2A = 80
CH = 32
ET1 = EP // 16
ET2 = EP // 32
NCH1 = ET1 // CH
NCH2 = ET2 // CH
RT = NP // 16

_SC_PARAMS = dict(compiler_params=pltpu.CompilerParams(use_tc_tiling_on_sc=False))


def _stage_a(z, W1, A1s, A1d):
    BR = 1000

    def body(z_ref, w_ref, s_ref, d_ref, xw_ref, as_ref, ad_ref):
        xw = jnp.dot(z_ref[...], w_ref[...], preferred_element_type=jnp.float32)
        xw_ref[...] = xw
        as_ref[...] = jnp.dot(xw, s_ref[...], preferred_element_type=jnp.float32)
        ad_ref[...] = jnp.dot(xw, d_ref[...], preferred_element_type=jnp.float32)

    return pl.pallas_call(
        body,
        grid=(10,),
        in_specs=[
            pl.BlockSpec((BR, 256), lambda i: (i, 0)),
            pl.BlockSpec((256, 1024), lambda i: (0, 0)),
            pl.BlockSpec((1024, 8), lambda i: (0, 0)),
            pl.BlockSpec((1024, 8), lambda i: (0, 0)),
        ],
        out_specs=[
            pl.BlockSpec((BR, 1024), lambda i: (i, 0)),
            pl.BlockSpec((BR, 8), lambda i: (i, 0)),
            pl.BlockSpec((BR, 8), lambda i: (i, 0)),
        ],
        out_shape=[
            jax.ShapeDtypeStruct((N, 1024), jnp.float32),
            jax.ShapeDtypeStruct((N, 8), jnp.float32),
            jax.ShapeDtypeStruct((N, 8), jnp.float32),
        ],
    )(z, W1, A1s, A1d)


def _colmax_lrelu(a, b):
    def body(a_ref, b_ref, o_ref):
        m = (jnp.max(a_ref[...], axis=0, keepdims=True)
             + jnp.max(b_ref[...], axis=0, keepdims=True))
        o_ref[...] = jnp.maximum(m, 0.2 * m)

    d = a.shape[1]
    return pl.pallas_call(
        body, out_shape=jax.ShapeDtypeStruct((1, d), jnp.float32))(a, b)


def _sc_scratch(da, et):
    tys = [pltpu.VMEM((et,), jnp.int32), pltpu.VMEM((et,), jnp.int32)]
    for _ in range(3):
        tys += [
            pltpu.VMEM((CH,), jnp.int32),
            pltpu.VMEM((CH,), jnp.int32),
            pltpu.VMEM((CH,), jnp.int32),
            pltpu.VMEM((CH, da), jnp.float32),
            pltpu.VMEM((CH,), jnp.float32),
            pltpu.VMEM((CH,), jnp.float32),
            pltpu.SemaphoreType.DMA,
            pltpu.SemaphoreType.DMA,
        ]
    tys.append(pltpu.VMEM((16,), jnp.float32))
    tys.append(pltpu.VMEM_SHARED((NP, da), jnp.float32))
    return tys


def _mk_pipeline(da, nch, table_h, asf_h, adf_h, z_h, acc, srcv, dstv, bufs,
                 cvec, off):
    nv = da // 16

    def issue(b, c):
        (gidx, didx, sidx, rows, asg, adg, sg, ss) = bufs[b]
        co = c * CH
        for j2 in range(CH // 16):
            sv = srcv[pl.ds(co + j2 * 16, 16)]
            dv = dstv[pl.ds(co + j2 * 16, 16)]
            gidx[pl.ds(j2 * 16, 16)] = sv + off
            didx[pl.ds(j2 * 16, 16)] = dv + off
            sidx[pl.ds(j2 * 16, 16)] = dv
        pltpu.async_copy(table_h.at[gidx], rows, sg)
        pltpu.async_copy(asf_h.at[gidx], asg, sg)
        pltpu.async_copy(adf_h.at[didx], adg, sg)

    def wait_g(b):
        (gidx, didx, sidx, rows, asg, adg, sg, ss) = bufs[b]
        pltpu.make_async_copy(table_h.at[pl.ds(0, CH)], rows, sg).wait()
        pltpu.make_async_copy(asf_h.at[pl.ds(0, CH)], asg, sg).wait()
        pltpu.make_async_copy(adf_h.at[pl.ds(0, CH)], adg, sg).wait()

    def compute(b, wait_before_scatter):
        (gidx, didx, sidx, rows, asg, adg, sg, ss) = bufs[b]
        for j2 in range(CH // 16):
            e = asg[pl.ds(j2 * 16, 16)] + adg[pl.ds(j2 * 16, 16)]
            w = jnp.exp(jnp.maximum(e, 0.2 * e) - cvec)
            for j in range(16):
                wv = jnp.full((16,), w[j], jnp.float32)
                jj = j2 * 16 + j
                for r in range(nv - 1):
                    sl = pl.ds(r * 16, 16)
                    rows[jj, sl] = rows[jj, sl] * wv
                rows[jj, pl.ds((nv - 1) * 16, 16)] = wv
        wait_before_scatter()
        pltpu.sync_copy(rows, acc.at[sidx], add=True)

    def wait_s(b):
        del b

    def run():
        def body(i, carry):
            c0 = 3 * i

            @pl.when(i == 0)
            def _():
                issue(0, c0)
                issue(1, c0 + 1)

            @pl.when(i > 0)
            def _():
                wait_s(2)

            issue(2, c0 + 2)
            wait_g(0)
            compute(0, lambda: None)
            wait_g(1)
            compute(1, lambda: wait_s(0))

            @pl.when(i < nch // 3 - 1)
            def _():
                issue(0, c0 + 3)

            wait_g(2)
            compute(2, lambda: wait_s(1))

            @pl.when(i < nch // 3 - 1)
            def _():
                issue(1, c0 + 4)

            return carry

        lax.fori_loop(0, nch // 3, body, 0)
        wait_s(2)

    return run


def _sc_gat1(table, asf, adf, cb, src, dst, zeros1):
    mesh = plsc.VectorSubcoreMesh(core_axis_name="c", subcore_axis_name="s")

    @functools.partial(
        pl.kernel,
        out_type=jax.ShapeDtypeStruct((H1, NP, D1A), jnp.float32),
        mesh=mesh,
        scratch_types=_sc_scratch(D1A, ET1),
        **_SC_PARAMS,
    )
    def k(table_h, asf_h, adf_h, cb_h, src_h, dst_h, z_h, out_h,
          srcv, dstv, *scr):
        bufs = [scr[i * 8:(i + 1) * 8] for i in range(3)]
        cbv, acc = scr[24], scr[25]
        cid = lax.axis_index("c")
        sid = lax.axis_index("s")
        ebase = sid * ET1
        rbase = sid * RT
        pltpu.sync_copy(src_h.at[pl.ds(ebase, ET1)], srcv)
        pltpu.sync_copy(dst_h.at[pl.ds(ebase, ET1)], dstv)

        def head(hh, carry):
            ah = cid * 4 + hh
            pltpu.sync_copy(z_h.at[pl.ds(rbase, RT)], acc.at[pl.ds(rbase, RT)])
            pltpu.sync_copy(cb_h.at[ah], cbv)
            plsc.subcore_barrier()
            cvec = cbv[...]
            hoff = jnp.full((16,), ah * NP, jnp.int32)
            _mk_pipeline(D1A, NCH1, table_h, asf_h, adf_h, z_h, acc,
                         srcv, dstv, bufs, cvec, hoff)()
            plsc.subcore_barrier()
            pltpu.sync_copy(acc.at[pl.ds(rbase, RT)],
                            out_h.at[ah].at[pl.ds(rbase, RT)])
            return carry

        lax.fori_loop(0, 4, head, 0)

    return k(table, asf, adf, cb, src, dst, zeros1)


def _stage_c(acc1, W2, b1r, a2s, a2d):
    BR = NP // 4

    def body(x_ref, w2_ref, b1_ref, s_ref, d_ref, xw2_ref, as_ref, ad_ref):
        h = pl.program_id(1)
        x = x_ref[0]
        den = x[:, 128:129]
        den = jnp.where(den == 0.0, 1.0, den)
        h1 = x[:, :128] / den + b1_ref[0]
        h1 = jnp.where(h1 > 0, h1, jnp.exp(h1) - 1.0)
        part = jnp.dot(h1, w2_ref[0], preferred_element_type=jnp.float32)

        @pl.when(h == 0)
        def _():
            xw2_ref[...] = part

        @pl.when(h > 0)
        def _():
            xw2_ref[...] += part

        @pl.when(h == H1 - 1)
        def _():
            xw2 = xw2_ref[...]
            as_ref[...] = jnp.dot(xw2, s_ref[...],
                                  preferred_element_type=jnp.float32)
            ad_ref[...] = jnp.dot(xw2, d_ref[...],
                                  preferred_element_type=jnp.float32)

    return pl.pallas_call(
        body,
        grid=(NP // BR, H1),
        in_specs=[
            pl.BlockSpec((1, BR, D1A), lambda i, h: (h, i, 0)),
            pl.BlockSpec((1, 128, 64), lambda i, h: (h, 0, 0)),
            pl.BlockSpec((1, 1, 128), lambda i, h: (h, 0, 0)),
            pl.BlockSpec((64, 8), lambda i, h: (0, 0)),
            pl.BlockSpec((64, 8), lambda i, h: (0, 0)),
        ],
        out_specs=[
            pl.BlockSpec((BR, 64), lambda i, h: (i, 0)),
            pl.BlockSpec((BR, 8), lambda i, h: (i, 0)),
            pl.BlockSpec((BR, 8), lambda i, h: (i, 0)),
        ],
        out_shape=[
            jax.ShapeDtypeStruct((NP, 64), jnp.float32),
            jax.ShapeDtypeStruct((NP, 8), jnp.float32),
            jax.ShapeDtypeStruct((NP, 8), jnp.float32),
        ],
    )(acc1, W2, b1r, a2s, a2d)


def _sc_gat2(table, asf, adf, cb, src, dst, zeros2):
    mesh = plsc.VectorSubcoreMesh(core_axis_name="c", subcore_axis_name="s")

    @functools.partial(
        pl.kernel,
        out_type=jax.ShapeDtypeStruct((2, NP, D2A), jnp.float32),
        mesh=mesh,
        scratch_types=_sc_scratch(D2A, ET2),
        **_SC_PARAMS,
    )
    def k(table_h, asf_h, adf_h, cb_h, src_h, dst_h, z_h, out_h,
          srcv, dstv, *scr):
        bufs = [scr[i * 8:(i + 1) * 8] for i in range(3)]
        cbv, acc = scr[24], scr[25]
        cid = lax.axis_index("c")
        sid = lax.axis_index("s")
        wid = cid * 16 + sid
        ebase = wid * ET2
        rbase = sid * RT
        pltpu.sync_copy(src_h.at[pl.ds(ebase, ET2)], srcv)
        pltpu.sync_copy(dst_h.at[pl.ds(ebase, ET2)], dstv)
        pltpu.sync_copy(z_h.at[pl.ds(rbase, RT)], acc.at[pl.ds(rbase, RT)])
        pltpu.sync_copy(cb_h, cbv)
        plsc.subcore_barrier()
        zoff = jnp.zeros((16,), jnp.int32)
        _mk_pipeline(D2A, NCH2, table_h, asf_h, adf_h, z_h, acc,
                     srcv, dstv, bufs, cbv[...], zoff)()
        plsc.subcore_barrier()
        pltpu.sync_copy(acc.at[pl.ds(rbase, RT)],
                        out_h.at[cid].at[pl.ds(rbase, RT)])

    return k(table, asf, adf, cb, src, dst, zeros2)


def _stage_e(p0, p1, b2r, cluster):
    BR = NP // 4

    def body(p0_ref, p1_ref, b2_ref, cl_ref, h_ref, q_ref):
        acc = p0_ref[...] + p1_ref[...]
        den = acc[:, 64:65]
        den = jnp.where(den == 0.0, 1.0, den)
        h = acc[:, :64] / den + b2_ref[...]
        h_ref[...] = h
        cl = cl_ref[...]
        hc = lax.dot_general(h, cl, (((1,), (1,)), ((), ())),
                             preferred_element_type=jnp.float32)
        h2 = jnp.sum(h * h, axis=1, keepdims=True)
        c2 = jnp.sum(cl * cl, axis=1)[None, :]
        q0 = 1.0 / (1.0 + h2 - 2.0 * hc + c2)
        q_ref[...] = q0 / jnp.sum(q0, axis=1, keepdims=True)

    return pl.pallas_call(
        body,
        grid=(NP // BR,),
        in_specs=[
            pl.BlockSpec((BR, D2A), lambda i: (i, 0)),
            pl.BlockSpec((BR, D2A), lambda i: (i, 0)),
            pl.BlockSpec((1, 64), lambda i: (0, 0)),
            pl.BlockSpec((16, 64), lambda i: (0, 0)),
        ],
        out_specs=[
            pl.BlockSpec((BR, 64), lambda i: (i, 0)),
            pl.BlockSpec((BR, 16), lambda i: (i, 0)),
        ],
        out_shape=[
            jax.ShapeDtypeStruct((NP, 64), jnp.float32),
            jax.ShapeDtypeStruct((NP, 16), jnp.float32),
        ],
    )(p0, p1, b2r, cluster)


def kernel(z, edge_index, W1, a1_src, a1_dst, b1, W2, a2_src, a2_dst, b2,
           cluster):
    src = edge_index[0].astype(jnp.int32)
    dst = edge_index[1].astype(jnp.int32)
    loop = jnp.arange(N, dtype=jnp.int32)
    npad = EP - src.shape[0] - N
    srcp = jnp.concatenate([src, loop, jnp.zeros((npad,), jnp.int32)])
    dstp = jnp.concatenate([dst, loop, jnp.full((npad,), N, jnp.int32)])

    eye = jnp.eye(H1, dtype=jnp.float32)
    A1s = (eye[:, None, :] * a1_src[:, :, None]).reshape(H1 * O1, H1)
    A1d = (eye[:, None, :] * a1_dst[:, :, None]).reshape(H1 * O1, H1)

    xw1, as1, ad1 = _stage_a(z, W1, A1s, A1d)
    C1 = _colmax_lrelu(as1, ad1)

    xw1h = jnp.pad(xw1.reshape(N, H1, O1).transpose(1, 0, 2),
                   ((0, 0), (0, NP - N), (0, 0)))
    table1 = jnp.concatenate(
        [xw1h, jnp.ones((H1, NP, 1), jnp.float32),
         jnp.zeros((H1, NP, D1A - O1 - 1), jnp.float32)],
        axis=2).reshape(H1 * NP, D1A)
    asf1 = jnp.pad(as1.T, ((0, 0), (0, NP - N))).reshape(H1 * NP)
    adf1 = jnp.pad(ad1.T, ((0, 0), (0, NP - N))).reshape(H1 * NP)
    cb1 = jnp.broadcast_to(C1.reshape(H1, 1), (H1, 16))
    zeros1 = jnp.zeros((NP, D1A), jnp.float32)

    acc1 = _sc_gat1(table1, asf1, adf1, cb1, srcp, dstp, zeros1)

    W2h = W2.reshape(H1, O1, O2)
    b1r = b1.reshape(H1, 1, O1)
    a2s = jnp.pad(a2_src.T, ((0, 0), (0, 7)))
    a2d = jnp.pad(a2_dst.T, ((0, 0), (0, 7)))
    xw2, as2p, ad2p = _stage_c(acc1, W2h, b1r, a2s, a2d)
    as2 = as2p[:N, :1]
    ad2 = ad2p[:N, :1]
    C2 = _colmax_lrelu(as2, ad2)

    table2 = jnp.concatenate(
        [xw2, jnp.ones((NP, 1), jnp.float32),
         jnp.zeros((NP, D2A - O2 - 1), jnp.float32)], axis=1)
    as2v = jnp.pad(as2[:, 0], (0, NP - N))
    ad2v = jnp.pad(ad2[:, 0], (0, NP - N))
    cb2 = jnp.broadcast_to(C2.reshape(1), (16,))
    zeros2 = jnp.zeros((NP, D2A), jnp.float32)

    parts = _sc_gat2(table2, as2v, ad2v, cb2, srcp, dstp, zeros2)

    b2r = b2.reshape(1, O2)
    hpad, qpad = _stage_e(parts[0], parts[1], b2r, cluster)
    return (hpad[:N], qpad[:N])


# speedup vs baseline: 1.0178x; 1.0178x over previous
import functools

import jax
import jax.numpy as jnp
from jax import lax
from jax.experimental import pallas as pl
from jax.experimental.pallas import tpu as pltpu
from jax.experimental.pallas import tpu_sc as plsc

N = 10000
NP = 10112
EP = 172032
H1 = 8
O1 = 128
O2 = 64
D1A = 144
D2A = 80
CH = 32
ET1 = EP // 16
ET2 = EP // 32
NCH1 = ET1 // CH
NCH2 = ET2 // CH
RT = NP // 16

_SC_PARAMS = dict(compiler_params=pltpu.CompilerParams(use_tc_tiling_on_sc=False))


def _stage_a(z, W1, A1s, A1d):
    BR = 1000

    def body(z_ref, w_ref, s_ref, d_ref, tab_ref, as_ref, ad_ref,
             ms_ref, md_ref):
        i = pl.program_id(0)
        xw = jnp.dot(z_ref[...], w_ref[...], preferred_element_type=jnp.float32)
        as_ = jnp.dot(xw, s_ref[...], preferred_element_type=jnp.float32)
        ad_ = jnp.dot(xw, d_ref[...], preferred_element_type=jnp.float32)
        as_ref[...] = as_
        ad_ref[...] = ad_
        for h in range(H1):
            tab_ref[h, :, :O1] = xw[:, h * O1:(h + 1) * O1]
        tab_ref[:, :, O1:O1 + 1] = jnp.ones((H1, BR, 1), jnp.float32)
        tab_ref[:, :, O1 + 1:] = jnp.zeros((H1, BR, D1A - O1 - 1), jnp.float32)
        ms = jnp.broadcast_to(jnp.max(as_, axis=0, keepdims=True), (H1, H1))
        md = jnp.broadcast_to(jnp.max(ad_, axis=0, keepdims=True), (H1, H1))

        @pl.when(i == 0)
        def _():
            ms_ref[...] = ms
            md_ref[...] = md

        @pl.when(i > 0)
        def _():
            ms_ref[...] = jnp.maximum(ms_ref[...], ms)
            md_ref[...] = jnp.maximum(md_ref[...], md)

    return pl.pallas_call(
        body,
        grid=(10,),
        in_specs=[
            pl.BlockSpec((BR, 256), lambda i: (i, 0)),
            pl.BlockSpec((256, 1024), lambda i: (0, 0)),
            pl.BlockSpec((1024, 8), lambda i: (0, 0)),
            pl.BlockSpec((1024, 8), lambda i: (0, 0)),
        ],
        out_specs=[
            pl.BlockSpec((H1, BR, D1A), lambda i: (0, i, 0)),
            pl.BlockSpec((BR, H1), lambda i: (i, 0)),
            pl.BlockSpec((BR, H1), lambda i: (i, 0)),
            pl.BlockSpec((H1, H1), lambda i: (0, 0)),
            pl.BlockSpec((H1, H1), lambda i: (0, 0)),
        ],
        out_shape=[
            jax.ShapeDtypeStruct((H1, N, D1A), jnp.float32),
            jax.ShapeDtypeStruct((N, H1), jnp.float32),
            jax.ShapeDtypeStruct((N, H1), jnp.float32),
            jax.ShapeDtypeStruct((H1, H1), jnp.float32),
            jax.ShapeDtypeStruct((H1, H1), jnp.float32),
        ],
    )(z, W1, A1s, A1d)


def _colmax_sum(a, b):
    def body(a_ref, b_ref, o_ref):
        o_ref[...] = (jnp.max(a_ref[...], axis=0, keepdims=True)
                      + jnp.max(b_ref[...], axis=0, keepdims=True))

    d = a.shape[1]
    return pl.pallas_call(
        body, out_shape=jax.ShapeDtypeStruct((1, d), jnp.float32))(a, b)


def _sc_scratch(da, et):
    tys = [pltpu.VMEM((et,), jnp.int32), pltpu.VMEM((et,), jnp.int32)]
    for _ in range(3):
        tys += [
            pltpu.VMEM((CH,), jnp.int32),
            pltpu.VMEM((CH,), jnp.int32),
            pltpu.VMEM((CH,), jnp.int32),
            pltpu.VMEM((CH, da), jnp.float32),
            pltpu.VMEM((CH,), jnp.float32),
            pltpu.VMEM((CH,), jnp.float32),
            pltpu.SemaphoreType.DMA,
            pltpu.SemaphoreType.DMA,
        ]
    tys.append(pltpu.VMEM((16,), jnp.float32))
    tys.append(pltpu.VMEM_SHARED((NP, da), jnp.float32))
    return tys


def _mk_pipeline(da, nch, table_h, asf_h, adf_h, z_h, acc, srcv, dstv, bufs,
                 cvec, goff, doff):
    nv = da // 16

    def issue(b, c):
        (gidx, didx, sidx, rows, asg, adg, sg, ss) = bufs[b]
        co = c * CH
        for j2 in range(CH // 16):
            sv = srcv[pl.ds(co + j2 * 16, 16)]
            dv = dstv[pl.ds(co + j2 * 16, 16)]
            gidx[pl.ds(j2 * 16, 16)] = sv + goff
            didx[pl.ds(j2 * 16, 16)] = dv + doff
            sidx[pl.ds(j2 * 16, 16)] = dv
        pltpu.async_copy(table_h.at[gidx], rows, sg)
        pltpu.async_copy(asf_h.at[gidx], asg, sg)
        pltpu.async_copy(adf_h.at[didx], adg, sg)

    def wait_g(b):
        (gidx, didx, sidx, rows, asg, adg, sg, ss) = bufs[b]
        pltpu.make_async_copy(table_h.at[pl.ds(0, CH)], rows, sg).wait()
        pltpu.make_async_copy(asf_h.at[pl.ds(0, CH)], asg, sg).wait()
        pltpu.make_async_copy(adf_h.at[pl.ds(0, CH)], adg, sg).wait()

    def compute(b, wait_before_scatter):
        (gidx, didx, sidx, rows, asg, adg, sg, ss) = bufs[b]
        for j2 in range(CH // 16):
            e = asg[pl.ds(j2 * 16, 16)] + adg[pl.ds(j2 * 16, 16)]
            w = jnp.exp(jnp.maximum(e, 0.2 * e) - cvec)
            for j in range(16):
                wv = jnp.full((16,), w[j], jnp.float32)
                jj = j2 * 16 + j
                for r in range(nv - 1):
                    sl = pl.ds(r * 16, 16)
                    rows[jj, sl] = rows[jj, sl] * wv
                rows[jj, pl.ds((nv - 1) * 16, 16)] = wv
        wait_before_scatter()
        pltpu.sync_copy(rows, acc.at[sidx], add=True)

    def wait_s(b):
        del b

    def run():
        def body(i, carry):
            c0 = 3 * i

            @pl.when(i == 0)
            def _():
                issue(0, c0)
                issue(1, c0 + 1)

            @pl.when(i > 0)
            def _():
                wait_s(2)

            issue(2, c0 + 2)
            wait_g(0)
            compute(0, lambda: None)
            wait_g(1)
            compute(1, lambda: wait_s(0))

            @pl.when(i < nch // 3 - 1)
            def _():
                issue(0, c0 + 3)

            wait_g(2)
            compute(2, lambda: wait_s(1))

            @pl.when(i < nch // 3 - 1)
            def _():
                issue(1, c0 + 4)

            return carry

        lax.fori_loop(0, nch // 3, body, 0)
        wait_s(2)

    return run


def _sc_gat1(table, asf, adf, cb, src, dst, zeros1):
    mesh = plsc.VectorSubcoreMesh(core_axis_name="c", subcore_axis_name="s")

    @functools.partial(
        pl.kernel,
        out_type=jax.ShapeDtypeStruct((H1, NP, D1A), jnp.float32),
        mesh=mesh,
        scratch_types=_sc_scratch(D1A, ET1),
        **_SC_PARAMS,
    )
    def k(table_h, asf_h, adf_h, cb_h, src_h, dst_h, z_h, out_h,
          srcv, dstv, *scr):
        bufs = [scr[i * 8:(i + 1) * 8] for i in range(3)]
        cbv, acc = scr[24], scr[25]
        cid = lax.axis_index("c")
        sid = lax.axis_index("s")
        ebase = sid * ET1
        rbase = sid * RT
        pltpu.sync_copy(src_h.at[pl.ds(ebase, ET1)], srcv)
        pltpu.sync_copy(dst_h.at[pl.ds(ebase, ET1)], dstv)

        def head(hh, carry):
            ah = cid * 4 + hh
            pltpu.sync_copy(z_h.at[pl.ds(rbase, RT)], acc.at[pl.ds(rbase, RT)])
            pltpu.sync_copy(cb_h.at[ah], cbv)
            plsc.subcore_barrier()
            craw = cbv[...]
            cvec = jnp.maximum(craw, 0.2 * craw)
            goff = jnp.full((16,), ah * N, jnp.int32)
            doff = jnp.full((16,), ah * NP, jnp.int32)
            _mk_pipeline(D1A, NCH1, table_h, asf_h, adf_h, z_h, acc,
                         srcv, dstv, bufs, cvec, goff, doff)()
            plsc.subcore_barrier()
            pltpu.sync_copy(acc.at[pl.ds(rbase, RT)],
                            out_h.at[ah].at[pl.ds(rbase, RT)])
            return carry

        lax.fori_loop(0, 4, head, 0)

    return k(table, asf, adf, cb, src, dst, zeros1)


def _stage_c(acc1, W2, b1r, a2s, a2d):
    BR = NP // 4

    def body(x_ref, w2_ref, b1_ref, s_ref, d_ref, xw2_ref, as_ref, ad_ref):
        h = pl.program_id(1)
        x = x_ref[0]
        den = x[:, 128:129]
        den = jnp.where(den == 0.0, 1.0, den)
        h1 = x[:, :128] / den + b1_ref[0]
        h1 = jnp.where(h1 > 0, h1, jnp.exp(h1) - 1.0)
        part = jnp.dot(h1, w2_ref[0], preferred_element_type=jnp.float32)

        @pl.when(h == 0)
        def _():
            xw2_ref[...] = part

        @pl.when(h > 0)
        def _():
            xw2_ref[...] += part

        @pl.when(h == H1 - 1)
        def _():
            xw2 = xw2_ref[...]
            as_ref[...] = jnp.dot(xw2, s_ref[...],
                                  preferred_element_type=jnp.float32)
            ad_ref[...] = jnp.dot(xw2, d_ref[...],
                                  preferred_element_type=jnp.float32)

    return pl.pallas_call(
        body,
        grid=(NP // BR, H1),
        in_specs=[
            pl.BlockSpec((1, BR, D1A), lambda i, h: (h, i, 0)),
            pl.BlockSpec((1, 128, 64), lambda i, h: (h, 0, 0)),
            pl.BlockSpec((1, 1, 128), lambda i, h: (h, 0, 0)),
            pl.BlockSpec((64, 8), lambda i, h: (0, 0)),
            pl.BlockSpec((64, 8), lambda i, h: (0, 0)),
        ],
        out_specs=[
            pl.BlockSpec((BR, 64), lambda i, h: (i, 0)),
            pl.BlockSpec((BR, 8), lambda i, h: (i, 0)),
            pl.BlockSpec((BR, 8), lambda i, h: (i, 0)),
        ],
        out_shape=[
            jax.ShapeDtypeStruct((NP, 64), jnp.float32),
            jax.ShapeDtypeStruct((NP, 8), jnp.float32),
            jax.ShapeDtypeStruct((NP, 8), jnp.float32),
        ],
    )(acc1, W2, b1r, a2s, a2d)


def _sc_gat2(table, asf, adf, cb, src, dst, zeros2):
    mesh = plsc.VectorSubcoreMesh(core_axis_name="c", subcore_axis_name="s")

    @functools.partial(
        pl.kernel,
        out_type=jax.ShapeDtypeStruct((2, NP, D2A), jnp.float32),
        mesh=mesh,
        scratch_types=_sc_scratch(D2A, ET2),
        **_SC_PARAMS,
    )
    def k(table_h, asf_h, adf_h, cb_h, src_h, dst_h, z_h, out_h,
          srcv, dstv, *scr):
        bufs = [scr[i * 8:(i + 1) * 8] for i in range(3)]
        cbv, acc = scr[24], scr[25]
        cid = lax.axis_index("c")
        sid = lax.axis_index("s")
        wid = cid * 16 + sid
        ebase = wid * ET2
        rbase = sid * RT
        pltpu.sync_copy(src_h.at[pl.ds(ebase, ET2)], srcv)
        pltpu.sync_copy(dst_h.at[pl.ds(ebase, ET2)], dstv)
        pltpu.sync_copy(z_h.at[pl.ds(rbase, RT)], acc.at[pl.ds(rbase, RT)])
        pltpu.sync_copy(cb_h, cbv)
        plsc.subcore_barrier()
        zoff = jnp.zeros((16,), jnp.int32)
        craw = cbv[...]
        cvec = jnp.maximum(craw, 0.2 * craw)
        _mk_pipeline(D2A, NCH2, table_h, asf_h, adf_h, z_h, acc,
                     srcv, dstv, bufs, cvec, zoff, zoff)()
        plsc.subcore_barrier()
        pltpu.sync_copy(acc.at[pl.ds(rbase, RT)],
                        out_h.at[cid].at[pl.ds(rbase, RT)])

    return k(table, asf, adf, cb, src, dst, zeros2)


def _stage_e(p0, p1, b2r, cluster):
    BR = NP // 4

    def body(p0_ref, p1_ref, b2_ref, cl_ref, h_ref, q_ref):
        acc = p0_ref[...] + p1_ref[...]
        den = acc[:, 64:65]
        den = jnp.where(den == 0.0, 1.0, den)
        h = acc[:, :64] / den + b2_ref[...]
        h_ref[...] = h
        cl = cl_ref[...]
        hc = lax.dot_general(h, cl, (((1,), (1,)), ((), ())),
                             preferred_element_type=jnp.float32)
        h2 = jnp.sum(h * h, axis=1, keepdims=True)
        c2 = jnp.sum(cl * cl, axis=1)[None, :]
        q0 = 1.0 / (1.0 + h2 - 2.0 * hc + c2)
        q_ref[...] = q0 / jnp.sum(q0, axis=1, keepdims=True)

    return pl.pallas_call(
        body,
        grid=(NP // BR,),
        in_specs=[
            pl.BlockSpec((BR, D2A), lambda i: (i, 0)),
            pl.BlockSpec((BR, D2A), lambda i: (i, 0)),
            pl.BlockSpec((1, 64), lambda i: (0, 0)),
            pl.BlockSpec((16, 64), lambda i: (0, 0)),
        ],
        out_specs=[
            pl.BlockSpec((BR, 64), lambda i: (i, 0)),
            pl.BlockSpec((BR, 16), lambda i: (i, 0)),
        ],
        out_shape=[
            jax.ShapeDtypeStruct((NP, 64), jnp.float32),
            jax.ShapeDtypeStruct((NP, 16), jnp.float32),
        ],
    )(p0, p1, b2r, cluster)


def kernel(z, edge_index, W1, a1_src, a1_dst, b1, W2, a2_src, a2_dst, b2,
           cluster):
    src = edge_index[0].astype(jnp.int32)
    dst = edge_index[1].astype(jnp.int32)
    loop = jnp.arange(N, dtype=jnp.int32)
    npad = EP - src.shape[0] - N
    srcp = jnp.concatenate([src, loop, jnp.zeros((npad,), jnp.int32)])
    dstp = jnp.concatenate([dst, loop, jnp.full((npad,), N, jnp.int32)])

    eye = jnp.eye(H1, dtype=jnp.float32)
    A1s = (eye[:, None, :] * a1_src[:, :, None]).reshape(H1 * O1, H1)
    A1d = (eye[:, None, :] * a1_dst[:, :, None]).reshape(H1 * O1, H1)

    tab1, as1, ad1, ms1, md1 = _stage_a(z, W1, A1s, A1d)
    table1 = tab1.reshape(H1 * N, D1A)
    asf1 = as1.T.reshape(H1 * N)
    adf1 = jnp.pad(ad1.T, ((0, 0), (0, NP - N))).reshape(H1 * NP)
    cb1 = jnp.broadcast_to((ms1[0] + md1[0]).reshape(H1, 1), (H1, 16))
    zeros1 = jnp.zeros((NP, D1A), jnp.float32)

    acc1 = _sc_gat1(table1, asf1, adf1, cb1, srcp, dstp, zeros1)

    W2h = W2.reshape(H1, O1, O2)
    b1r = b1.reshape(H1, 1, O1)
    a2s = jnp.pad(a2_src.T, ((0, 0), (0, 7)))
    a2d = jnp.pad(a2_dst.T, ((0, 0), (0, 7)))
    xw2, as2p, ad2p = _stage_c(acc1, W2h, b1r, a2s, a2d)
    as2 = as2p[:N, :1]
    ad2 = ad2p[:N, :1]
    C2 = _colmax_sum(as2, ad2)

    table2 = jnp.concatenate(
        [xw2, jnp.ones((NP, 1), jnp.float32),
         jnp.zeros((NP, D2A - O2 - 1), jnp.float32)], axis=1)
    as2v = jnp.pad(as2[:, 0], (0, NP - N))
    ad2v = jnp.pad(ad2[:, 0], (0, NP - N))
    cb2 = jnp.broadcast_to(C2.reshape(1), (16,))
    zeros2 = jnp.zeros((NP, D2A), jnp.float32)

    parts = _sc_gat2(table2, as2v, ad2v, cb2, srcp, dstp, zeros2)

    b2r = b2.reshape(1, O2)
    hpad, qpad = _stage_e(parts[0], parts[1], b2r, cluster)
    return (hpad[:N], qpad[:N])


# speedup vs baseline: 1.0682x; 1.0496x over previous
import functools

import jax
import jax.numpy as jnp
from jax import lax
from jax.experimental import pallas as pl
from jax.experimental.pallas import tpu as pltpu
from jax.experimental.pallas import tpu_sc as plsc

N = 10000
NP = 10112
EP = 172032
H1 = 8
O1 = 128
O2 = 64
D1A = 144
D2A = 80
CH = 32
ET1 = EP // 16
ET2 = EP // 32
NCH1 = ET1 // CH
NCH2 = ET2 // CH
RT = NP // 16

_SC_PARAMS = dict(compiler_params=pltpu.CompilerParams(use_tc_tiling_on_sc=False))


def _stage_a(z, W1, A1s, A1d):
    BR = 1000

    def body(z_ref, w_ref, s_ref, d_ref, tab_ref, as_ref, ad_ref,
             ms_ref, md_ref):
        i = pl.program_id(0)
        xw = jnp.dot(z_ref[...], w_ref[...], preferred_element_type=jnp.float32)
        as_ = jnp.dot(xw, s_ref[...], preferred_element_type=jnp.float32)
        ad_ = jnp.dot(xw, d_ref[...], preferred_element_type=jnp.float32)
        as_ref[...] = as_
        ad_ref[...] = ad_
        for h in range(H1):
            tab_ref[h, :, :O1] = xw[:, h * O1:(h + 1) * O1]
        tab_ref[:, :, O1:O1 + 1] = jnp.ones((H1, BR, 1), jnp.float32)
        tab_ref[:, :, O1 + 1:] = jnp.zeros((H1, BR, D1A - O1 - 1), jnp.float32)
        ms = jnp.broadcast_to(jnp.max(as_, axis=0, keepdims=True), (H1, H1))
        md = jnp.broadcast_to(jnp.max(ad_, axis=0, keepdims=True), (H1, H1))

        @pl.when(i == 0)
        def _():
            ms_ref[...] = ms
            md_ref[...] = md

        @pl.when(i > 0)
        def _():
            ms_ref[...] = jnp.maximum(ms_ref[...], ms)
            md_ref[...] = jnp.maximum(md_ref[...], md)

    return pl.pallas_call(
        body,
        grid=(10,),
        in_specs=[
            pl.BlockSpec((BR, 256), lambda i: (i, 0)),
            pl.BlockSpec((256, 1024), lambda i: (0, 0)),
            pl.BlockSpec((1024, 8), lambda i: (0, 0)),
            pl.BlockSpec((1024, 8), lambda i: (0, 0)),
        ],
        out_specs=[
            pl.BlockSpec((H1, BR, D1A), lambda i: (0, i, 0)),
            pl.BlockSpec((BR, H1), lambda i: (i, 0)),
            pl.BlockSpec((BR, H1), lambda i: (i, 0)),
            pl.BlockSpec((H1, H1), lambda i: (0, 0)),
            pl.BlockSpec((H1, H1), lambda i: (0, 0)),
        ],
        out_shape=[
            jax.ShapeDtypeStruct((H1, N, D1A), jnp.float32),
            jax.ShapeDtypeStruct((N, H1), jnp.float32),
            jax.ShapeDtypeStruct((N, H1), jnp.float32),
            jax.ShapeDtypeStruct((H1, H1), jnp.float32),
            jax.ShapeDtypeStruct((H1, H1), jnp.float32),
        ],
    )(z, W1, A1s, A1d)


def _colmax_sum(a, b):
    def body(a_ref, b_ref, o_ref):
        o_ref[...] = (jnp.max(a_ref[...], axis=0, keepdims=True)
                      + jnp.max(b_ref[...], axis=0, keepdims=True))

    d = a.shape[1]
    return pl.pallas_call(
        body, out_shape=jax.ShapeDtypeStruct((1, d), jnp.float32))(a, b)


def _sc_scratch(da, et):
    tys = [pltpu.VMEM((et,), jnp.int32), pltpu.VMEM((et,), jnp.int32)]
    for _ in range(3):
        tys += [
            pltpu.VMEM((CH,), jnp.int32),
            pltpu.VMEM((CH,), jnp.int32),
            pltpu.VMEM((CH,), jnp.int32),
            pltpu.VMEM((CH, da), jnp.float32),
            pltpu.VMEM((CH,), jnp.float32),
            pltpu.VMEM((CH,), jnp.float32),
            pltpu.SemaphoreType.DMA,
            pltpu.SemaphoreType.DMA,
        ]
    tys.append(pltpu.VMEM((16,), jnp.float32))
    tys.append(pltpu.VMEM_SHARED((NP, da), jnp.float32))
    return tys


def _mk_pipeline(da, nch, table_h, asf_h, adf_h, z_h, acc, srcv, dstv, bufs,
                 cvec, goff, doff):
    nv = da // 16

    def issue(b, c):
        (gidx, didx, sidx, rows, asg, adg, sg, ss) = bufs[b]
        co = c * CH
        for j2 in range(CH // 16):
            sv = srcv[pl.ds(co + j2 * 16, 16)]
            dv = dstv[pl.ds(co + j2 * 16, 16)]
            gidx[pl.ds(j2 * 16, 16)] = sv + goff
            didx[pl.ds(j2 * 16, 16)] = dv + doff
            sidx[pl.ds(j2 * 16, 16)] = dv
        pltpu.async_copy(table_h.at[gidx], rows, sg)
        pltpu.async_copy(asf_h.at[gidx], asg, sg)
        pltpu.async_copy(adf_h.at[didx], adg, sg)

    def wait_g(b):
        (gidx, didx, sidx, rows, asg, adg, sg, ss) = bufs[b]
        pltpu.make_async_copy(table_h.at[pl.ds(0, CH)], rows, sg).wait()
        pltpu.make_async_copy(asf_h.at[pl.ds(0, CH)], asg, sg).wait()
        pltpu.make_async_copy(adf_h.at[pl.ds(0, CH)], adg, sg).wait()

    def scale(b):
        (gidx, didx, sidx, rows, asg, adg, sg, ss) = bufs[b]
        for j2 in range(CH // 16):
            e = asg[pl.ds(j2 * 16, 16)] + adg[pl.ds(j2 * 16, 16)]
            w = jnp.exp(jnp.maximum(e, 0.2 * e) - cvec)
            for j in range(16):
                wv = jnp.full((16,), w[j], jnp.float32)
                jj = j2 * 16 + j
                for r in range(nv - 1):
                    sl = pl.ds(r * 16, 16)
                    rows[jj, sl] = rows[jj, sl] * wv
                rows[jj, pl.ds((nv - 1) * 16, 16)] = wv

    def scatter_async(b):
        (gidx, didx, sidx, rows, asg, adg, sg, ss) = bufs[b]
        return pltpu.async_copy(rows, acc.at[sidx], ss, add=True)

    def scatter_sync(b):
        (gidx, didx, sidx, rows, asg, adg, sg, ss) = bufs[b]
        pltpu.sync_copy(rows, acc.at[sidx], add=True)

    def run():
        def body(i, carry):
            c0 = 3 * i

            @pl.when(i == 0)
            def _():
                issue(0, c0)
                issue(1, c0 + 1)

            issue(2, c0 + 2)
            wait_g(0)
            scale(0)
            h0 = scatter_async(0)
            wait_g(1)
            scale(1)
            h0.wait()
            h1 = scatter_async(1)

            @pl.when(i < nch // 3 - 1)
            def _():
                issue(0, c0 + 3)

            wait_g(2)
            scale(2)
            h1.wait()
            scatter_sync(2)

            @pl.when(i < nch // 3 - 1)
            def _():
                issue(1, c0 + 4)

            return carry

        lax.fori_loop(0, nch // 3, body, 0)

    return run


def _sc_gat1(table, asf, adf, cb, src, dst, zeros1):
    mesh = plsc.VectorSubcoreMesh(core_axis_name="c", subcore_axis_name="s")

    @functools.partial(
        pl.kernel,
        out_type=jax.ShapeDtypeStruct((H1, NP, D1A), jnp.float32),
        mesh=mesh,
        scratch_types=_sc_scratch(D1A, ET1),
        **_SC_PARAMS,
    )
    def k(table_h, asf_h, adf_h, cb_h, src_h, dst_h, z_h, out_h,
          srcv, dstv, *scr):
        bufs = [scr[i * 8:(i + 1) * 8] for i in range(3)]
        cbv, acc = scr[24], scr[25]
        cid = lax.axis_index("c")
        sid = lax.axis_index("s")
        ebase = sid * ET1
        rbase = sid * RT
        pltpu.sync_copy(src_h.at[pl.ds(ebase, ET1)], srcv)
        pltpu.sync_copy(dst_h.at[pl.ds(ebase, ET1)], dstv)

        def head(hh, carry):
            ah = cid * 4 + hh
            pltpu.sync_copy(z_h.at[pl.ds(rbase, RT)], acc.at[pl.ds(rbase, RT)])
            pltpu.sync_copy(cb_h.at[ah], cbv)
            plsc.subcore_barrier()
            craw = cbv[...]
            cvec = jnp.maximum(craw, 0.2 * craw)
            goff = jnp.full((16,), ah * N, jnp.int32)
            doff = jnp.full((16,), ah * NP, jnp.int32)
            _mk_pipeline(D1A, NCH1, table_h, asf_h, adf_h, z_h, acc,
                         srcv, dstv, bufs, cvec, goff, doff)()
            plsc.subcore_barrier()
            pltpu.sync_copy(acc.at[pl.ds(rbase, RT)],
                            out_h.at[ah].at[pl.ds(rbase, RT)])
            return carry

        lax.fori_loop(0, 4, head, 0)

    return k(table, asf, adf, cb, src, dst, zeros1)


def _stage_c(acc1, W2, b1r, a2s, a2d):
    BR = NP // 4

    def body(x_ref, w2_ref, b1_ref, s_ref, d_ref, xw2_ref, as_ref, ad_ref):
        h = pl.program_id(1)
        x = x_ref[0]
        den = x[:, 128:129]
        den = jnp.where(den == 0.0, 1.0, den)
        h1 = x[:, :128] / den + b1_ref[0]
        h1 = jnp.where(h1 > 0, h1, jnp.exp(h1) - 1.0)
        part = jnp.dot(h1, w2_ref[0], preferred_element_type=jnp.float32)

        @pl.when(h == 0)
        def _():
            xw2_ref[...] = part

        @pl.when(h > 0)
        def _():
            xw2_ref[...] += part

        @pl.when(h == H1 - 1)
        def _():
            xw2 = xw2_ref[...]
            as_ref[...] = jnp.dot(xw2, s_ref[...],
                                  preferred_element_type=jnp.float32)
            ad_ref[...] = jnp.dot(xw2, d_ref[...],
                                  preferred_element_type=jnp.float32)

    return pl.pallas_call(
        body,
        grid=(NP // BR, H1),
        in_specs=[
            pl.BlockSpec((1, BR, D1A), lambda i, h: (h, i, 0)),
            pl.BlockSpec((1, 128, 64), lambda i, h: (h, 0, 0)),
            pl.BlockSpec((1, 1, 128), lambda i, h: (h, 0, 0)),
            pl.BlockSpec((64, 8), lambda i, h: (0, 0)),
            pl.BlockSpec((64, 8), lambda i, h: (0, 0)),
        ],
        out_specs=[
            pl.BlockSpec((BR, 64), lambda i, h: (i, 0)),
            pl.BlockSpec((BR, 8), lambda i, h: (i, 0)),
            pl.BlockSpec((BR, 8), lambda i, h: (i, 0)),
        ],
        out_shape=[
            jax.ShapeDtypeStruct((NP, 64), jnp.float32),
            jax.ShapeDtypeStruct((NP, 8), jnp.float32),
            jax.ShapeDtypeStruct((NP, 8), jnp.float32),
        ],
    )(acc1, W2, b1r, a2s, a2d)


def _sc_gat2(table, asf, adf, cb, src, dst, zeros2):
    mesh = plsc.VectorSubcoreMesh(core_axis_name="c", subcore_axis_name="s")

    @functools.partial(
        pl.kernel,
        out_type=jax.ShapeDtypeStruct((2, NP, D2A), jnp.float32),
        mesh=mesh,
        scratch_types=_sc_scratch(D2A, ET2),
        **_SC_PARAMS,
    )
    def k(table_h, asf_h, adf_h, cb_h, src_h, dst_h, z_h, out_h,
          srcv, dstv, *scr):
        bufs = [scr[i * 8:(i + 1) * 8] for i in range(3)]
        cbv, acc = scr[24], scr[25]
        cid = lax.axis_index("c")
        sid = lax.axis_index("s")
        wid = cid * 16 + sid
        ebase = wid * ET2
        rbase = sid * RT
        pltpu.sync_copy(src_h.at[pl.ds(ebase, ET2)], srcv)
        pltpu.sync_copy(dst_h.at[pl.ds(ebase, ET2)], dstv)
        pltpu.sync_copy(z_h.at[pl.ds(rbase, RT)], acc.at[pl.ds(rbase, RT)])
        pltpu.sync_copy(cb_h, cbv)
        plsc.subcore_barrier()
        zoff = jnp.zeros((16,), jnp.int32)
        craw = cbv[...]
        cvec = jnp.maximum(craw, 0.2 * craw)
        _mk_pipeline(D2A, NCH2, table_h, asf_h, adf_h, z_h, acc,
                     srcv, dstv, bufs, cvec, zoff, zoff)()
        plsc.subcore_barrier()
        pltpu.sync_copy(acc.at[pl.ds(rbase, RT)],
                        out_h.at[cid].at[pl.ds(rbase, RT)])

    return k(table, asf, adf, cb, src, dst, zeros2)


def _stage_e(p0, p1, b2r, cluster):
    BR = NP // 4

    def body(p0_ref, p1_ref, b2_ref, cl_ref, h_ref, q_ref):
        acc = p0_ref[...] + p1_ref[...]
        den = acc[:, 64:65]
        den = jnp.where(den == 0.0, 1.0, den)
        h = acc[:, :64] / den + b2_ref[...]
        h_ref[...] = h
        cl = cl_ref[...]
        hc = lax.dot_general(h, cl, (((1,), (1,)), ((), ())),
                             preferred_element_type=jnp.float32)
        h2 = jnp.sum(h * h, axis=1, keepdims=True)
        c2 = jnp.sum(cl * cl, axis=1)[None, :]
        q0 = 1.0 / (1.0 + h2 - 2.0 * hc + c2)
        q_ref[...] = q0 / jnp.sum(q0, axis=1, keepdims=True)

    return pl.pallas_call(
        body,
        grid=(NP // BR,),
        in_specs=[
            pl.BlockSpec((BR, D2A), lambda i: (i, 0)),
            pl.BlockSpec((BR, D2A), lambda i: (i, 0)),
            pl.BlockSpec((1, 64), lambda i: (0, 0)),
            pl.BlockSpec((16, 64), lambda i: (0, 0)),
        ],
        out_specs=[
            pl.BlockSpec((BR, 64), lambda i: (i, 0)),
            pl.BlockSpec((BR, 16), lambda i: (i, 0)),
        ],
        out_shape=[
            jax.ShapeDtypeStruct((NP, 64), jnp.float32),
            jax.ShapeDtypeStruct((NP, 16), jnp.float32),
        ],
    )(p0, p1, b2r, cluster)


def kernel(z, edge_index, W1, a1_src, a1_dst, b1, W2, a2_src, a2_dst, b2,
           cluster):
    src = edge_index[0].astype(jnp.int32)
    dst = edge_index[1].astype(jnp.int32)
    loop = jnp.arange(N, dtype=jnp.int32)
    npad = EP - src.shape[0] - N
    srcp = jnp.concatenate([src, loop, jnp.zeros((npad,), jnp.int32)])
    dstp = jnp.concatenate([dst, loop, jnp.full((npad,), N, jnp.int32)])

    eye = jnp.eye(H1, dtype=jnp.float32)
    A1s = (eye[:, None, :] * a1_src[:, :, None]).reshape(H1 * O1, H1)
    A1d = (eye[:, None, :] * a1_dst[:, :, None]).reshape(H1 * O1, H1)

    tab1, as1, ad1, ms1, md1 = _stage_a(z, W1, A1s, A1d)
    table1 = tab1.reshape(H1 * N, D1A)
    asf1 = as1.T.reshape(H1 * N)
    adf1 = jnp.pad(ad1.T, ((0, 0), (0, NP - N))).reshape(H1 * NP)
    cb1 = jnp.broadcast_to((ms1[0] + md1[0]).reshape(H1, 1), (H1, 16))
    zeros1 = jnp.zeros((NP, D1A), jnp.float32)

    acc1 = _sc_gat1(table1, asf1, adf1, cb1, srcp, dstp, zeros1)

    W2h = W2.reshape(H1, O1, O2)
    b1r = b1.reshape(H1, 1, O1)
    a2s = jnp.pad(a2_src.T, ((0, 0), (0, 7)))
    a2d = jnp.pad(a2_dst.T, ((0, 0), (0, 7)))
    xw2, as2p, ad2p = _stage_c(acc1, W2h, b1r, a2s, a2d)
    as2 = as2p[:N, :1]
    ad2 = ad2p[:N, :1]
    C2 = _colmax_sum(as2, ad2)

    table2 = jnp.concatenate(
        [xw2, jnp.ones((NP, 1), jnp.float32),
         jnp.zeros((NP, D2A - O2 - 1), jnp.float32)], axis=1)
    as2v = jnp.pad(as2[:, 0], (0, NP - N))
    ad2v = jnp.pad(ad2[:, 0], (0, NP - N))
    cb2 = jnp.broadcast_to(C2.reshape(1), (16,))
    zeros2 = jnp.zeros((NP, D2A), jnp.float32)

    parts = _sc_gat2(table2, as2v, ad2v, cb2, srcp, dstp, zeros2)

    b2r = b2.reshape(1, O2)
    hpad, qpad = _stage_e(parts[0], parts[1], b2r, cluster)
    return (hpad[:N], qpad[:N])


# speedup vs baseline: 1.0906x; 1.0209x over previous
import functools

import jax
import jax.numpy as jnp
from jax import lax
from jax.experimental import pallas as pl
from jax.experimental.pallas import tpu as pltpu
from jax.experimental.pallas import tpu_sc as plsc

N = 10000
NP = 10112
EP = 172032
H1 = 8
O1 = 128
O2 = 64
D1A = 144
D2A = 80
CH = 64
ET1 = EP // 16
ET2 = EP // 32
NCH1 = ET1 // CH
NCH2 = ET2 // CH
RT = NP // 16

_SC_PARAMS = dict(compiler_params=pltpu.CompilerParams(use_tc_tiling_on_sc=False))


def _stage_a(z, W1, A1s, A1d):
    BR = 1000

    def body(z_ref, w_ref, s_ref, d_ref, tab_ref, as_ref, ad_ref,
             ms_ref, md_ref):
        i = pl.program_id(0)
        xw = jnp.dot(z_ref[...], w_ref[...], preferred_element_type=jnp.float32)
        as_ = jnp.dot(xw, s_ref[...], preferred_element_type=jnp.float32)
        ad_ = jnp.dot(xw, d_ref[...], preferred_element_type=jnp.float32)
        as_ref[...] = as_
        ad_ref[...] = ad_
        for h in range(H1):
            tab_ref[h, :, :O1] = xw[:, h * O1:(h + 1) * O1]
        tab_ref[:, :, O1:O1 + 1] = jnp.ones((H1, BR, 1), jnp.float32)
        tab_ref[:, :, O1 + 1:] = jnp.zeros((H1, BR, D1A - O1 - 1), jnp.float32)
        ms = jnp.broadcast_to(jnp.max(as_, axis=0, keepdims=True), (H1, H1))
        md = jnp.broadcast_to(jnp.max(ad_, axis=0, keepdims=True), (H1, H1))

        @pl.when(i == 0)
        def _():
            ms_ref[...] = ms
            md_ref[...] = md

        @pl.when(i > 0)
        def _():
            ms_ref[...] = jnp.maximum(ms_ref[...], ms)
            md_ref[...] = jnp.maximum(md_ref[...], md)

    return pl.pallas_call(
        body,
        grid=(10,),
        in_specs=[
            pl.BlockSpec((BR, 256), lambda i: (i, 0)),
            pl.BlockSpec((256, 1024), lambda i: (0, 0)),
            pl.BlockSpec((1024, 8), lambda i: (0, 0)),
            pl.BlockSpec((1024, 8), lambda i: (0, 0)),
        ],
        out_specs=[
            pl.BlockSpec((H1, BR, D1A), lambda i: (0, i, 0)),
            pl.BlockSpec((BR, H1), lambda i: (i, 0)),
            pl.BlockSpec((BR, H1), lambda i: (i, 0)),
            pl.BlockSpec((H1, H1), lambda i: (0, 0)),
            pl.BlockSpec((H1, H1), lambda i: (0, 0)),
        ],
        out_shape=[
            jax.ShapeDtypeStruct((H1, N, D1A), jnp.float32),
            jax.ShapeDtypeStruct((N, H1), jnp.float32),
            jax.ShapeDtypeStruct((N, H1), jnp.float32),
            jax.ShapeDtypeStruct((H1, H1), jnp.float32),
            jax.ShapeDtypeStruct((H1, H1), jnp.float32),
        ],
    )(z, W1, A1s, A1d)


def _colmax_sum(a, b):
    def body(a_ref, b_ref, o_ref):
        o_ref[...] = (jnp.max(a_ref[...], axis=0, keepdims=True)
                      + jnp.max(b_ref[...], axis=0, keepdims=True))

    d = a.shape[1]
    return pl.pallas_call(
        body, out_shape=jax.ShapeDtypeStruct((1, d), jnp.float32))(a, b)


def _sc_scratch(da, et):
    tys = [pltpu.VMEM((et,), jnp.int32)]
    for _ in range(3):
        tys += [
            pltpu.VMEM((CH,), jnp.int32),
            pltpu.VMEM((CH,), jnp.int32),
            pltpu.VMEM((CH,), jnp.int32),
            pltpu.VMEM((CH, da), jnp.float32),
            pltpu.VMEM((CH,), jnp.float32),
            pltpu.VMEM((CH,), jnp.float32),
            pltpu.SemaphoreType.DMA,
            pltpu.SemaphoreType.DMA,
        ]
    tys.append(pltpu.VMEM((16,), jnp.float32))
    tys.append(pltpu.VMEM_SHARED((NP, da), jnp.float32))
    return tys


def _mk_pipeline(da, nch, table_h, asf_h, adf_h, z_h, acc, sdv, bufs,
                 cvec, goff, doff):
    nv = da // 16

    def issue(b, c):
        (gidx, didx, sidx, rows, asg, adg, sg, ss) = bufs[b]
        co = c * CH
        for j2 in range(CH // 16):
            pv = sdv[pl.ds(co + j2 * 16, 16)]
            sv = lax.shift_right_logical(pv, 14)
            dv = jnp.bitwise_and(pv, 16383)
            gidx[pl.ds(j2 * 16, 16)] = sv + goff
            didx[pl.ds(j2 * 16, 16)] = dv + doff
            sidx[pl.ds(j2 * 16, 16)] = dv
        pltpu.async_copy(table_h.at[gidx], rows, sg)
        pltpu.async_copy(asf_h.at[gidx], asg, sg)
        pltpu.async_copy(adf_h.at[didx], adg, sg)

    def wait_g(b):
        (gidx, didx, sidx, rows, asg, adg, sg, ss) = bufs[b]
        pltpu.make_async_copy(table_h.at[pl.ds(0, CH)], rows, sg).wait()
        pltpu.make_async_copy(asf_h.at[pl.ds(0, CH)], asg, sg).wait()
        pltpu.make_async_copy(adf_h.at[pl.ds(0, CH)], adg, sg).wait()

    def scale(b):
        (gidx, didx, sidx, rows, asg, adg, sg, ss) = bufs[b]
        for j2 in range(CH // 16):
            e = asg[pl.ds(j2 * 16, 16)] + adg[pl.ds(j2 * 16, 16)]
            w = jnp.exp(jnp.maximum(e, 0.2 * e) - cvec)
            for j in range(16):
                wv = jnp.full((16,), w[j], jnp.float32)
                jj = j2 * 16 + j
                for r in range(nv - 1):
                    sl = pl.ds(r * 16, 16)
                    rows[jj, sl] = rows[jj, sl] * wv
                rows[jj, pl.ds((nv - 1) * 16, 16)] = wv

    def scatter_async(b):
        (gidx, didx, sidx, rows, asg, adg, sg, ss) = bufs[b]
        return pltpu.async_copy(rows, acc.at[sidx], ss, add=True)

    def scatter_sync(b):
        (gidx, didx, sidx, rows, asg, adg, sg, ss) = bufs[b]
        pltpu.sync_copy(rows, acc.at[sidx], add=True)

    def run():
        def body(i, carry):
            c0 = 3 * i

            @pl.when(i == 0)
            def _():
                issue(0, c0)
                issue(1, c0 + 1)

            issue(2, c0 + 2)
            wait_g(0)
            scale(0)
            h0 = scatter_async(0)
            wait_g(1)
            scale(1)
            h0.wait()
            h1 = scatter_async(1)

            @pl.when(i < nch // 3 - 1)
            def _():
                issue(0, c0 + 3)

            wait_g(2)
            scale(2)
            h1.wait()
            scatter_sync(2)

            @pl.when(i < nch // 3 - 1)
            def _():
                issue(1, c0 + 4)

            return carry

        lax.fori_loop(0, nch // 3, body, 0)

    return run


def _sc_gat1(table, asf, adf, cb, srcdst, zeros1):
    mesh = plsc.VectorSubcoreMesh(core_axis_name="c", subcore_axis_name="s")

    @functools.partial(
        pl.kernel,
        out_type=jax.ShapeDtypeStruct((H1, NP, D1A), jnp.float32),
        mesh=mesh,
        scratch_types=_sc_scratch(D1A, ET1),
        **_SC_PARAMS,
    )
    def k(table_h, asf_h, adf_h, cb_h, sd_h, z_h, out_h, sdv, *scr):
        bufs = [scr[i * 8:(i + 1) * 8] for i in range(3)]
        cbv, acc = scr[24], scr[25]
        cid = lax.axis_index("c")
        sid = lax.axis_index("s")
        ebase = sid * ET1
        rbase = sid * RT
        pltpu.sync_copy(sd_h.at[pl.ds(ebase, ET1)], sdv)

        def head(hh, carry):
            ah = cid * 4 + hh
            pltpu.sync_copy(z_h.at[pl.ds(rbase, RT)], acc.at[pl.ds(rbase, RT)])
            pltpu.sync_copy(cb_h.at[ah], cbv)
            plsc.subcore_barrier()
            craw = cbv[...]
            cvec = jnp.maximum(craw, 0.2 * craw)
            goff = jnp.full((16,), ah * N, jnp.int32)
            doff = jnp.full((16,), ah * NP, jnp.int32)
            _mk_pipeline(D1A, NCH1, table_h, asf_h, adf_h, z_h, acc,
                         sdv, bufs, cvec, goff, doff)()
            plsc.subcore_barrier()
            pltpu.sync_copy(acc.at[pl.ds(rbase, RT)],
                            out_h.at[ah].at[pl.ds(rbase, RT)])
            return carry

        lax.fori_loop(0, 4, head, 0)

    return k(table, asf, adf, cb, srcdst, zeros1)


def _stage_c(acc1, W2, b1r, a2s, a2d):
    BR = NP // 4

    def body(x_ref, w2_ref, b1_ref, s_ref, d_ref, xw2_ref, as_ref, ad_ref):
        h = pl.program_id(1)
        x = x_ref[0]
        den = x[:, 128:129]
        den = jnp.where(den == 0.0, 1.0, den)
        h1 = x[:, :128] / den + b1_ref[0]
        h1 = jnp.where(h1 > 0, h1, jnp.exp(h1) - 1.0)
        part = jnp.dot(h1, w2_ref[0], preferred_element_type=jnp.float32)

        @pl.when(h == 0)
        def _():
            xw2_ref[...] = part

        @pl.when(h > 0)
        def _():
            xw2_ref[...] += part

        @pl.when(h == H1 - 1)
        def _():
            xw2 = xw2_ref[...]
            as_ref[...] = jnp.dot(xw2, s_ref[...],
                                  preferred_element_type=jnp.float32)
            ad_ref[...] = jnp.dot(xw2, d_ref[...],
                                  preferred_element_type=jnp.float32)

    return pl.pallas_call(
        body,
        grid=(NP // BR, H1),
        in_specs=[
            pl.BlockSpec((1, BR, D1A), lambda i, h: (h, i, 0)),
            pl.BlockSpec((1, 128, 64), lambda i, h: (h, 0, 0)),
            pl.BlockSpec((1, 1, 128), lambda i, h: (h, 0, 0)),
            pl.BlockSpec((64, 8), lambda i, h: (0, 0)),
            pl.BlockSpec((64, 8), lambda i, h: (0, 0)),
        ],
        out_specs=[
            pl.BlockSpec((BR, 64), lambda i, h: (i, 0)),
            pl.BlockSpec((BR, 8), lambda i, h: (i, 0)),
            pl.BlockSpec((BR, 8), lambda i, h: (i, 0)),
        ],
        out_shape=[
            jax.ShapeDtypeStruct((NP, 64), jnp.float32),
            jax.ShapeDtypeStruct((NP, 8), jnp.float32),
            jax.ShapeDtypeStruct((NP, 8), jnp.float32),
        ],
    )(acc1, W2, b1r, a2s, a2d)


def _sc_gat2(table, asf, adf, cb, srcdst, zeros2):
    mesh = plsc.VectorSubcoreMesh(core_axis_name="c", subcore_axis_name="s")

    @functools.partial(
        pl.kernel,
        out_type=jax.ShapeDtypeStruct((2, NP, D2A), jnp.float32),
        mesh=mesh,
        scratch_types=_sc_scratch(D2A, ET2),
        **_SC_PARAMS,
    )
    def k(table_h, asf_h, adf_h, cb_h, sd_h, z_h, out_h, sdv, *scr):
        bufs = [scr[i * 8:(i + 1) * 8] for i in range(3)]
        cbv, acc = scr[24], scr[25]
        cid = lax.axis_index("c")
        sid = lax.axis_index("s")
        wid = cid * 16 + sid
        ebase = wid * ET2
        rbase = sid * RT
        pltpu.sync_copy(sd_h.at[pl.ds(ebase, ET2)], sdv)
        pltpu.sync_copy(z_h.at[pl.ds(rbase, RT)], acc.at[pl.ds(rbase, RT)])
        pltpu.sync_copy(cb_h, cbv)
        plsc.subcore_barrier()
        zoff = jnp.zeros((16,), jnp.int32)
        craw = cbv[...]
        cvec = jnp.maximum(craw, 0.2 * craw)
        _mk_pipeline(D2A, NCH2, table_h, asf_h, adf_h, z_h, acc,
                     sdv, bufs, cvec, zoff, zoff)()
        plsc.subcore_barrier()
        pltpu.sync_copy(acc.at[pl.ds(rbase, RT)],
                        out_h.at[cid].at[pl.ds(rbase, RT)])

    return k(table, asf, adf, cb, srcdst, zeros2)


def _stage_e(p0, p1, b2r, cluster):
    BR = NP // 4

    def body(p0_ref, p1_ref, b2_ref, cl_ref, h_ref, q_ref):
        acc = p0_ref[...] + p1_ref[...]
        den = acc[:, 64:65]
        den = jnp.where(den == 0.0, 1.0, den)
        h = acc[:, :64] / den + b2_ref[...]
        h_ref[...] = h
        cl = cl_ref[...]
        hc = lax.dot_general(h, cl, (((1,), (1,)), ((), ())),
                             preferred_element_type=jnp.float32)
        h2 = jnp.sum(h * h, axis=1, keepdims=True)
        c2 = jnp.sum(cl * cl, axis=1)[None, :]
        q0 = 1.0 / (1.0 + h2 - 2.0 * hc + c2)
        q_ref[...] = q0 / jnp.sum(q0, axis=1, keepdims=True)

    return pl.pallas_call(
        body,
        grid=(NP // BR,),
        in_specs=[
            pl.BlockSpec((BR, D2A), lambda i: (i, 0)),
            pl.BlockSpec((BR, D2A), lambda i: (i, 0)),
            pl.BlockSpec((1, 64), lambda i: (0, 0)),
            pl.BlockSpec((16, 64), lambda i: (0, 0)),
        ],
        out_specs=[
            pl.BlockSpec((BR, 64), lambda i: (i, 0)),
            pl.BlockSpec((BR, 16), lambda i: (i, 0)),
        ],
        out_shape=[
            jax.ShapeDtypeStruct((NP, 64), jnp.float32),
            jax.ShapeDtypeStruct((NP, 16), jnp.float32),
        ],
    )(p0, p1, b2r, cluster)


def kernel(z, edge_index, W1, a1_src, a1_dst, b1, W2, a2_src, a2_dst, b2,
           cluster):
    src = edge_index[0].astype(jnp.int32)
    dst = edge_index[1].astype(jnp.int32)
    loop = jnp.arange(N, dtype=jnp.int32)
    npad = EP - src.shape[0] - N
    srcp = jnp.concatenate([src, loop, jnp.zeros((npad,), jnp.int32)])
    dstp = jnp.concatenate([dst, loop, jnp.full((npad,), N, jnp.int32)])
    srcdst = srcp * 16384 + dstp

    eye = jnp.eye(H1, dtype=jnp.float32)
    A1s = (eye[:, None, :] * a1_src[:, :, None]).reshape(H1 * O1, H1)
    A1d = (eye[:, None, :] * a1_dst[:, :, None]).reshape(H1 * O1, H1)

    tab1, as1, ad1, ms1, md1 = _stage_a(z, W1, A1s, A1d)
    table1 = tab1.reshape(H1 * N, D1A)
    asf1 = as1.T.reshape(H1 * N)
    adf1 = jnp.pad(ad1.T, ((0, 0), (0, NP - N))).reshape(H1 * NP)
    cb1 = jnp.broadcast_to((ms1[0] + md1[0]).reshape(H1, 1), (H1, 16))
    zeros1 = jnp.zeros((NP, D1A), jnp.float32)

    acc1 = _sc_gat1(table1, asf1, adf1, cb1, srcdst, zeros1)

    W2h = W2.reshape(H1, O1, O2)
    b1r = b1.reshape(H1, 1, O1)
    a2s = jnp.pad(a2_src.T, ((0, 0), (0, 7)))
    a2d = jnp.pad(a2_dst.T, ((0, 0), (0, 7)))
    xw2, as2p, ad2p = _stage_c(acc1, W2h, b1r, a2s, a2d)
    as2 = as2p[:N, :1]
    ad2 = ad2p[:N, :1]
    C2 = _colmax_sum(as2, ad2)

    table2 = jnp.concatenate(
        [xw2, jnp.ones((NP, 1), jnp.float32),
         jnp.zeros((NP, D2A - O2 - 1), jnp.float32)], axis=1)
    as2v = jnp.pad(as2[:, 0], (0, NP - N))
    ad2v = jnp.pad(ad2[:, 0], (0, NP - N))
    cb2 = jnp.broadcast_to(C2.reshape(1), (16,))
    zeros2 = jnp.zeros((NP, D2A), jnp.float32)

    parts = _sc_gat2(table2, as2v, ad2v, cb2, srcdst, zeros2)

    b2r = b2.reshape(1, O2)
    hpad, qpad = _stage_e(parts[0], parts[1], b2r, cluster)
    return (hpad[:N], qpad[:N])


# speedup vs baseline: 1.1445x; 1.0495x over previous
import functools

import jax
import jax.numpy as jnp
from jax import lax
from jax.experimental import pallas as pl
from jax.experimental.pallas import tpu as pltpu
from jax.experimental.pallas import tpu_sc as plsc

N = 10000
NP = 10112
EP = 172032
H1 = 8
O1 = 128
O2 = 64
D1A = 144
D2A = 80
CH = 64
ET1 = EP // 16
ET2 = EP // 32
NCH1 = ET1 // CH
NCH2 = ET2 // CH
RT = NP // 16

_SC_PARAMS = dict(compiler_params=pltpu.CompilerParams(use_tc_tiling_on_sc=False))


def _stage_a(z, W1, A1s, A1d):
    BR = 1000

    def body(z_ref, w_ref, s_ref, d_ref, tab_ref, as_ref, ad_ref,
             ms_ref, md_ref):
        i = pl.program_id(0)
        xw = jnp.dot(z_ref[...], w_ref[...], preferred_element_type=jnp.float32)
        as_ = jnp.dot(xw, s_ref[...], preferred_element_type=jnp.float32)
        ad_ = jnp.dot(xw, d_ref[...], preferred_element_type=jnp.float32)
        as_ref[...] = as_
        ad_ref[...] = ad_
        for h in range(H1):
            tab_ref[h, :, :O1] = xw[:, h * O1:(h + 1) * O1]
        tab_ref[:, :, O1:O1 + 1] = jnp.ones((H1, BR, 1), jnp.float32)
        tab_ref[:, :, O1 + 1:] = jnp.zeros((H1, BR, D1A - O1 - 1), jnp.float32)
        ms = jnp.broadcast_to(jnp.max(as_, axis=0, keepdims=True), (H1, H1))
        md = jnp.broadcast_to(jnp.max(ad_, axis=0, keepdims=True), (H1, H1))

        @pl.when(i == 0)
        def _():
            ms_ref[...] = ms
            md_ref[...] = md

        @pl.when(i > 0)
        def _():
            ms_ref[...] = jnp.maximum(ms_ref[...], ms)
            md_ref[...] = jnp.maximum(md_ref[...], md)

    return pl.pallas_call(
        body,
        grid=(10,),
        in_specs=[
            pl.BlockSpec((BR, 256), lambda i: (i, 0)),
            pl.BlockSpec((256, 1024), lambda i: (0, 0)),
            pl.BlockSpec((1024, 8), lambda i: (0, 0)),
            pl.BlockSpec((1024, 8), lambda i: (0, 0)),
        ],
        out_specs=[
            pl.BlockSpec((H1, BR, D1A), lambda i: (0, i, 0)),
            pl.BlockSpec((BR, H1), lambda i: (i, 0)),
            pl.BlockSpec((BR, H1), lambda i: (i, 0)),
            pl.BlockSpec((H1, H1), lambda i: (0, 0)),
            pl.BlockSpec((H1, H1), lambda i: (0, 0)),
        ],
        out_shape=[
            jax.ShapeDtypeStruct((H1, N, D1A), jnp.float32),
            jax.ShapeDtypeStruct((N, H1), jnp.float32),
            jax.ShapeDtypeStruct((N, H1), jnp.float32),
            jax.ShapeDtypeStruct((H1, H1), jnp.float32),
            jax.ShapeDtypeStruct((H1, H1), jnp.float32),
        ],
    )(z, W1, A1s, A1d)


def _colmax_sum(a, b):
    def body(a_ref, b_ref, o_ref):
        o_ref[...] = (jnp.max(a_ref[...], axis=0, keepdims=True)
                      + jnp.max(b_ref[...], axis=0, keepdims=True))

    d = a.shape[1]
    return pl.pallas_call(
        body, out_shape=jax.ShapeDtypeStruct((1, d), jnp.float32))(a, b)


def _sc_scratch(da, et):
    tys = [pltpu.VMEM((et,), jnp.int32)]
    for _ in range(3):
        tys += [
            pltpu.VMEM((CH,), jnp.int32),
            pltpu.VMEM((CH,), jnp.int32),
            pltpu.VMEM((CH,), jnp.int32),
            pltpu.VMEM((CH, da), jnp.float32),
            pltpu.VMEM((CH,), jnp.float32),
            pltpu.VMEM((CH,), jnp.float32),
            pltpu.SemaphoreType.DMA,
            pltpu.SemaphoreType.DMA,
        ]
    tys.append(pltpu.VMEM((16,), jnp.float32))
    tys.append(pltpu.VMEM_SHARED((NP, da), jnp.float32))
    return tys


def _mk_pipeline(da, nch, table_h, asf_h, adf_h, z_h, acc, sdv, bufs,
                 cvec, goff, doff):
    nv = da // 16

    def issue(b, c):
        (gidx, didx, sidx, rows, asg, adg, sg, ss) = bufs[b]
        co = c * CH
        for j2 in range(CH // 16):
            pv = sdv[pl.ds(co + j2 * 16, 16)]
            sv = lax.shift_right_logical(pv, 14)
            dv = jnp.bitwise_and(pv, 16383)
            gidx[pl.ds(j2 * 16, 16)] = sv + goff
            didx[pl.ds(j2 * 16, 16)] = dv + doff
            sidx[pl.ds(j2 * 16, 16)] = dv
        pltpu.async_copy(table_h.at[gidx], rows, sg)
        pltpu.async_copy(asf_h.at[gidx], asg, sg)
        pltpu.async_copy(adf_h.at[didx], adg, sg)

    def wait_g(b):
        (gidx, didx, sidx, rows, asg, adg, sg, ss) = bufs[b]
        pltpu.make_async_copy(table_h.at[pl.ds(0, CH)], rows, sg).wait()
        pltpu.make_async_copy(asf_h.at[pl.ds(0, CH)], asg, sg).wait()
        pltpu.make_async_copy(adf_h.at[pl.ds(0, CH)], adg, sg).wait()

    def scale(b):
        (gidx, didx, sidx, rows, asg, adg, sg, ss) = bufs[b]
        for j2 in range(CH // 16):
            e = asg[pl.ds(j2 * 16, 16)] + adg[pl.ds(j2 * 16, 16)]
            w = jnp.exp(jnp.maximum(e, 0.2 * e) - cvec)
            for j in range(16):
                wv = jnp.full((16,), w[j], jnp.float32)
                jj = j2 * 16 + j
                for r in range(nv - 1):
                    sl = pl.ds(r * 16, 16)
                    rows[jj, sl] = rows[jj, sl] * wv
                rows[jj, pl.ds((nv - 1) * 16, 16)] = wv

    def scatter_async(b):
        (gidx, didx, sidx, rows, asg, adg, sg, ss) = bufs[b]
        return pltpu.async_copy(rows, acc.at[sidx], ss, add=True)

    def scatter_wait(b):
        (gidx, didx, sidx, rows, asg, adg, sg, ss) = bufs[b]
        pltpu.make_async_copy(rows, acc.at[sidx], ss).wait()

    def run():
        def body(i, carry):
            c0 = 3 * i

            @pl.when(i == 0)
            def _():
                issue(0, c0)
                issue(1, c0 + 1)

            @pl.when(i > 0)
            def _():
                scatter_wait(2)

            issue(2, c0 + 2)
            wait_g(0)
            scale(0)
            h0 = scatter_async(0)
            wait_g(1)
            scale(1)
            h0.wait()
            h1 = scatter_async(1)

            @pl.when(i < nch // 3 - 1)
            def _():
                issue(0, c0 + 3)

            wait_g(2)
            scale(2)
            h1.wait()
            scatter_async(2)

            @pl.when(i < nch // 3 - 1)
            def _():
                issue(1, c0 + 4)

            return carry

        lax.fori_loop(0, nch // 3, body, 0)
        scatter_wait(2)

    return run


def _sc_gat1(table, asf, adf, cb, srcdst, zeros1):
    mesh = plsc.VectorSubcoreMesh(core_axis_name="c", subcore_axis_name="s")

    @functools.partial(
        pl.kernel,
        out_type=jax.ShapeDtypeStruct((H1, NP, D1A), jnp.float32),
        mesh=mesh,
        scratch_types=_sc_scratch(D1A, ET1),
        **_SC_PARAMS,
    )
    def k(table_h, asf_h, adf_h, cb_h, sd_h, z_h, out_h, sdv, *scr):
        bufs = [scr[i * 8:(i + 1) * 8] for i in range(3)]
        cbv, acc = scr[24], scr[25]
        cid = lax.axis_index("c")
        sid = lax.axis_index("s")
        ebase = sid * ET1
        rbase = sid * RT
        pltpu.sync_copy(sd_h.at[pl.ds(ebase, ET1)], sdv)

        def head(hh, carry):
            ah = cid * 4 + hh
            pltpu.sync_copy(z_h.at[pl.ds(rbase, RT)], acc.at[pl.ds(rbase, RT)])
            pltpu.sync_copy(cb_h.at[ah], cbv)
            plsc.subcore_barrier()
            craw = cbv[...]
            cvec = jnp.maximum(craw, 0.2 * craw)
            goff = jnp.full((16,), ah * N, jnp.int32)
            doff = jnp.full((16,), ah * NP, jnp.int32)
            _mk_pipeline(D1A, NCH1, table_h, asf_h, adf_h, z_h, acc,
                         sdv, bufs, cvec, goff, doff)()
            plsc.subcore_barrier()
            pltpu.sync_copy(acc.at[pl.ds(rbase, RT)],
                            out_h.at[ah].at[pl.ds(rbase, RT)])
            return carry

        lax.fori_loop(0, 4, head, 0)

    return k(table, asf, adf, cb, srcdst, zeros1)


def _stage_c(acc1, W2, b1r, a2s, a2d):
    BR = NP // 4

    def body(x_ref, w2_ref, b1_ref, s_ref, d_ref, xw2_ref, as_ref, ad_ref):
        h = pl.program_id(1)
        x = x_ref[0]
        den = x[:, 128:129]
        den = jnp.where(den == 0.0, 1.0, den)
        h1 = x[:, :128] / den + b1_ref[0]
        h1 = jnp.where(h1 > 0, h1, jnp.exp(h1) - 1.0)
        part = jnp.dot(h1, w2_ref[0], preferred_element_type=jnp.float32)

        @pl.when(h == 0)
        def _():
            xw2_ref[...] = part

        @pl.when(h > 0)
        def _():
            xw2_ref[...] += part

        @pl.when(h == H1 - 1)
        def _():
            xw2 = xw2_ref[...]
            as_ref[...] = jnp.dot(xw2, s_ref[...],
                                  preferred_element_type=jnp.float32)
            ad_ref[...] = jnp.dot(xw2, d_ref[...],
                                  preferred_element_type=jnp.float32)

    return pl.pallas_call(
        body,
        grid=(NP // BR, H1),
        in_specs=[
            pl.BlockSpec((1, BR, D1A), lambda i, h: (h, i, 0)),
            pl.BlockSpec((1, 128, 64), lambda i, h: (h, 0, 0)),
            pl.BlockSpec((1, 1, 128), lambda i, h: (h, 0, 0)),
            pl.BlockSpec((64, 8), lambda i, h: (0, 0)),
            pl.BlockSpec((64, 8), lambda i, h: (0, 0)),
        ],
        out_specs=[
            pl.BlockSpec((BR, 64), lambda i, h: (i, 0)),
            pl.BlockSpec((BR, 8), lambda i, h: (i, 0)),
            pl.BlockSpec((BR, 8), lambda i, h: (i, 0)),
        ],
        out_shape=[
            jax.ShapeDtypeStruct((NP, 64), jnp.float32),
            jax.ShapeDtypeStruct((NP, 8), jnp.float32),
            jax.ShapeDtypeStruct((NP, 8), jnp.float32),
        ],
    )(acc1, W2, b1r, a2s, a2d)


def _sc_gat2(table, asf, adf, cb, srcdst, zeros2):
    mesh = plsc.VectorSubcoreMesh(core_axis_name="c", subcore_axis_name="s")

    @functools.partial(
        pl.kernel,
        out_type=jax.ShapeDtypeStruct((2, NP, D2A), jnp.float32),
        mesh=mesh,
        scratch_types=_sc_scratch(D2A, ET2),
        **_SC_PARAMS,
    )
    def k(table_h, asf_h, adf_h, cb_h, sd_h, z_h, out_h, sdv, *scr):
        bufs = [scr[i * 8:(i + 1) * 8] for i in range(3)]
        cbv, acc = scr[24], scr[25]
        cid = lax.axis_index("c")
        sid = lax.axis_index("s")
        wid = cid * 16 + sid
        ebase = wid * ET2
        rbase = sid * RT
        pltpu.sync_copy(sd_h.at[pl.ds(ebase, ET2)], sdv)
        pltpu.sync_copy(z_h.at[pl.ds(rbase, RT)], acc.at[pl.ds(rbase, RT)])
        pltpu.sync_copy(cb_h, cbv)
        plsc.subcore_barrier()
        zoff = jnp.zeros((16,), jnp.int32)
        craw = cbv[...]
        cvec = jnp.maximum(craw, 0.2 * craw)
        _mk_pipeline(D2A, NCH2, table_h, asf_h, adf_h, z_h, acc,
                     sdv, bufs, cvec, zoff, zoff)()
        plsc.subcore_barrier()
        pltpu.sync_copy(acc.at[pl.ds(rbase, RT)],
                        out_h.at[cid].at[pl.ds(rbase, RT)])

    return k(table, asf, adf, cb, srcdst, zeros2)


def _stage_e(p0, p1, b2r, cluster):
    BR = NP // 4

    def body(p0_ref, p1_ref, b2_ref, cl_ref, h_ref, q_ref):
        acc = p0_ref[...] + p1_ref[...]
        den = acc[:, 64:65]
        den = jnp.where(den == 0.0, 1.0, den)
        h = acc[:, :64] / den + b2_ref[...]
        h_ref[...] = h
        cl = cl_ref[...]
        hc = lax.dot_general(h, cl, (((1,), (1,)), ((), ())),
                             preferred_element_type=jnp.float32)
        h2 = jnp.sum(h * h, axis=1, keepdims=True)
        c2 = jnp.sum(cl * cl, axis=1)[None, :]
        q0 = 1.0 / (1.0 + h2 - 2.0 * hc + c2)
        q_ref[...] = q0 / jnp.sum(q0, axis=1, keepdims=True)

    return pl.pallas_call(
        body,
        grid=(NP // BR,),
        in_specs=[
            pl.BlockSpec((BR, D2A), lambda i: (i, 0)),
            pl.BlockSpec((BR, D2A), lambda i: (i, 0)),
            pl.BlockSpec((1, 64), lambda i: (0, 0)),
            pl.BlockSpec((16, 64), lambda i: (0, 0)),
        ],
        out_specs=[
            pl.BlockSpec((BR, 64), lambda i: (i, 0)),
            pl.BlockSpec((BR, 16), lambda i: (i, 0)),
        ],
        out_shape=[
            jax.ShapeDtypeStruct((NP, 64), jnp.float32),
            jax.ShapeDtypeStruct((NP, 16), jnp.float32),
        ],
    )(p0, p1, b2r, cluster)


def kernel(z, edge_index, W1, a1_src, a1_dst, b1, W2, a2_src, a2_dst, b2,
           cluster):
    src = edge_index[0].astype(jnp.int32)
    dst = edge_index[1].astype(jnp.int32)
    loop = jnp.arange(N, dtype=jnp.int32)
    npad = EP - src.shape[0] - N
    srcp = jnp.concatenate([src, loop, jnp.zeros((npad,), jnp.int32)])
    dstp = jnp.concatenate([dst, loop, jnp.full((npad,), N, jnp.int32)])
    srcdst = srcp * 16384 + dstp

    eye = jnp.eye(H1, dtype=jnp.float32)
    A1s = (eye[:, None, :] * a1_src[:, :, None]).reshape(H1 * O1, H1)
    A1d = (eye[:, None, :] * a1_dst[:, :, None]).reshape(H1 * O1, H1)

    tab1, as1, ad1, ms1, md1 = _stage_a(z, W1, A1s, A1d)
    table1 = tab1.reshape(H1 * N, D1A)
    asf1 = as1.T.reshape(H1 * N)
    adf1 = jnp.pad(ad1.T, ((0, 0), (0, NP - N))).reshape(H1 * NP)
    cb1 = jnp.broadcast_to((ms1[0] + md1[0]).reshape(H1, 1), (H1, 16))
    zeros1 = jnp.zeros((NP, D1A), jnp.float32)

    acc1 = _sc_gat1(table1, asf1, adf1, cb1, srcdst, zeros1)

    W2h = W2.reshape(H1, O1, O2)
    b1r = b1.reshape(H1, 1, O1)
    a2s = jnp.pad(a2_src.T, ((0, 0), (0, 7)))
    a2d = jnp.pad(a2_dst.T, ((0, 0), (0, 7)))
    xw2, as2p, ad2p = _stage_c(acc1, W2h, b1r, a2s, a2d)
    as2 = as2p[:N, :1]
    ad2 = ad2p[:N, :1]
    C2 = _colmax_sum(as2, ad2)

    table2 = jnp.concatenate(
        [xw2, jnp.ones((NP, 1), jnp.float32),
         jnp.zeros((NP, D2A - O2 - 1), jnp.float32)], axis=1)
    as2v = jnp.pad(as2[:, 0], (0, NP - N))
    ad2v = jnp.pad(ad2[:, 0], (0, NP - N))
    cb2 = jnp.broadcast_to(C2.reshape(1), (16,))
    zeros2 = jnp.zeros((NP, D2A), jnp.float32)

    parts = _sc_gat2(table2, as2v, ad2v, cb2, srcdst, zeros2)

    b2r = b2.reshape(1, O2)
    hpad, qpad = _stage_e(parts[0], parts[1], b2r, cluster)
    return (hpad[:N], qpad[:N])
